# Initial kernel scaffold; baseline (speedup 1.0000x reference)
import jax
import jax.numpy as jnp
from jax.experimental import pallas as pl


def kernel(x, adj, edge, weight, W_ih, W_hh, b_ih, b_hh):
    raise NotImplementedError("write your pallas kernel here")


# speedup vs baseline: 2.3336x; 2.3336x over previous
import jax
import jax.numpy as jnp
from jax import lax
from jax.experimental import pallas as pl
from jax.experimental.pallas import tpu as pltpu
from jax.experimental.pallas import tpu_sc as plsc

N = 10000
D = 128
NC = 2
NS = 16
NW = NC * NS
CHUNK = 128
KI = 8
OUTER = 10
EDGES_PER_W = OUTER * KI * CHUNK
E_PAD = NW * EDGES_PER_W
N_PAD = 10240
ROWS_PER_TILE = N_PAD // NS
DUMMY = N


def _sc_scatter_body(h_hbm, src_hbm, dst_hbm, out_hbm,
                     acc, src_st, dst_st, rows, zbuf, sem):
    cid = lax.axis_index("c")
    sid = lax.axis_index("s")
    wid = cid * NS + sid
    r0 = sid * ROWS_PER_TILE
    zero16 = jnp.zeros((16,), jnp.float32)

    def zfill(r, _):
        for c in range(D // 16):
            zbuf[r, pl.ds(c * 16, 16)] = zero16
        return 0
    lax.fori_loop(0, CHUNK, zfill, 0)
    for k in range(ROWS_PER_TILE // CHUNK):
        pltpu.sync_copy(zbuf, acc.at[pl.ds(r0 + k * CHUNK, CHUNK)])

    plsc.subcore_barrier()

    def step(t, _):
        pltpu.sync_copy(src_hbm.at[wid, pl.ds(t * KI, KI)], src_st)
        pltpu.sync_copy(dst_hbm.at[wid, pl.ds(t * KI, KI)], dst_st)
        for j in range(KI):
            pltpu.async_copy(h_hbm.at[src_st.at[j]], rows, sem).wait()
            pltpu.sync_copy(rows, acc.at[dst_st.at[j]], add=True)
        return 0
    lax.fori_loop(0, OUTER, step, 0)

    plsc.subcore_barrier()

    for k in range(ROWS_PER_TILE // CHUNK):
        s = pl.ds(r0 + k * CHUNK, CHUNK)
        pltpu.sync_copy(acc.at[s], out_hbm.at[cid, s])


import functools


@functools.cache
def _get_sc_scatter():
    return pl.kernel(
        _sc_scatter_body,
        out_type=jax.ShapeDtypeStruct((NC, N_PAD, D), jnp.float32),
        mesh=plsc.VectorSubcoreMesh(
            core_axis_name="c", subcore_axis_name="s",
            num_cores=NC, num_subcores=NS),
        scratch_types=[
            pltpu.VMEM_SHARED((N_PAD, D), jnp.float32),
            pltpu.VMEM((KI, CHUNK), jnp.int32),
            pltpu.VMEM((KI, CHUNK), jnp.int32),
            pltpu.VMEM((CHUNK, D), jnp.float32),
            pltpu.VMEM((CHUNK, D), jnp.float32),
            pltpu.SemaphoreType.DMA,
        ],
        name="gcn_sc_scatter",
    )


RB = 1000


def _tc_body(p_ref, deg_ref, h_ref, w_ref, wihT_ref, whhT_ref, bih_ref,
             bhh_ref, o_ref):
    hp = jax.lax.Precision.HIGHEST
    g = p_ref[0] + p_ref[1]
    deg = deg_ref[0, :, 0:1] + deg_ref[1, :, 0:1]
    deg = jnp.maximum(deg, 1.0)
    agg = jax.lax.dot(g / deg, w_ref[...], precision=hp)
    gi = jax.lax.dot(agg, wihT_ref[...], precision=hp) + bih_ref[...]
    h = h_ref[...]
    gh = jax.lax.dot(h, whhT_ref[...], precision=hp) + bhh_ref[...]
    r = jax.nn.sigmoid(gi[:, :D] + gh[:, :D])
    z = jax.nn.sigmoid(gi[:, D:2 * D] + gh[:, D:2 * D])
    n = jnp.tanh(gi[:, 2 * D:] + r * gh[:, 2 * D:])
    o_ref[...] = (1.0 - z) * n + z * h


def _tc_layer(p, degp, h, w, wihT, whhT, bih, bhh):
    return pl.pallas_call(
        _tc_body,
        grid=(N // RB,),
        in_specs=[
            pl.BlockSpec((NC, RB, D), lambda i: (0, i, 0)),
            pl.BlockSpec((NC, RB, D), lambda i: (0, i, 0)),
            pl.BlockSpec((RB, D), lambda i: (i, 0)),
            pl.BlockSpec((D, D), lambda i: (0, 0)),
            pl.BlockSpec((D, 3 * D), lambda i: (0, 0)),
            pl.BlockSpec((D, 3 * D), lambda i: (0, 0)),
            pl.BlockSpec((1, 3 * D), lambda i: (0, 0)),
            pl.BlockSpec((1, 3 * D), lambda i: (0, 0)),
        ],
        out_specs=pl.BlockSpec((RB, D), lambda i: (i, 0)),
        out_shape=jax.ShapeDtypeStruct((N, D), jnp.float32),
    )(p, degp, h, w, wihT, whhT, bih, bhh)


def kernel(x, adj, edge, weight, W_ih, W_hh, b_ih, b_hh):
    src = edge[0].astype(jnp.int32)
    dst = edge[1].astype(jnp.int32)
    e = src.shape[0]
    pad = E_PAD - e
    src_p = jnp.concatenate([src, jnp.zeros((pad,), jnp.int32)])
    dst_p = jnp.concatenate([dst, jnp.full((pad,), DUMMY, jnp.int32)])
    src3 = src_p.reshape(NW, OUTER * KI, CHUNK)
    dst3 = dst_p.reshape(NW, OUTER * KI, CHUNK)
    wihT = W_ih.T
    whhT = W_hh.T
    bih2 = b_ih.reshape(1, 3 * D)
    bhh2 = b_hh.reshape(1, 3 * D)

    degp = _get_sc_scatter()(jnp.ones((N, D), jnp.float32), src3, dst3)
    h = x
    for i in range(3):
        p = _get_sc_scatter()(h, src3, dst3)
        h = _tc_layer(p, degp, h, weight[i], wihT, whhT, bih2, bhh2)
    return h


# speedup vs baseline: 3.0727x; 1.3168x over previous
import functools

import jax
import jax.numpy as jnp
from jax import lax
from jax.experimental import pallas as pl
from jax.experimental.pallas import tpu as pltpu
from jax.experimental.pallas import tpu_sc as plsc

N = 10000
D = 128
NC = 2
NS = 16
NW = NC * NS
CHUNK = 128
KI = 16
OUTER = 5
EDGES_PER_W = OUTER * KI * CHUNK
E_PAD = NW * EDGES_PER_W
N_PAD = 10240
ROWS_PER_TILE = N_PAD // NS
DUMMY = N


def _sc_scatter_body(h_hbm, src_hbm, dst_hbm, out_hbm,
                     acc, src_st, dst_st, rows_a, rows_b, sem_a, sem_b):
    cid = lax.axis_index("c")
    sid = lax.axis_index("s")
    wid = cid * NS + sid
    r0 = sid * ROWS_PER_TILE
    zero16 = jnp.zeros((16,), jnp.float32)

    def zfill(r, _):
        for c in range(D // 16):
            rows_a[r, pl.ds(c * 16, 16)] = zero16
        return 0
    lax.fori_loop(0, CHUNK, zfill, 0)
    for k in range(ROWS_PER_TILE // CHUNK):
        pltpu.sync_copy(rows_a, acc.at[pl.ds(r0 + k * CHUNK, CHUNK)])

    plsc.subcore_barrier()

    bufs = (rows_a, rows_b)
    sems = (sem_a, sem_b)

    def step(t, _):
        pltpu.sync_copy(src_hbm.at[wid, pl.ds(t * KI, KI)], src_st)
        pltpu.sync_copy(dst_hbm.at[wid, pl.ds(t * KI, KI)], dst_st)
        pltpu.async_copy(h_hbm.at[src_st.at[0]], bufs[0], sems[0])
        for j in range(KI):
            cur = j % 2
            nxt = 1 - cur
            pltpu.make_async_copy(h_hbm.at[src_st.at[j]], bufs[cur],
                                  sems[cur]).wait()
            if j + 1 < KI:
                pltpu.async_copy(h_hbm.at[src_st.at[j + 1]], bufs[nxt],
                                 sems[nxt])
            pltpu.sync_copy(bufs[cur], acc.at[dst_st.at[j]], add=True)
        return 0
    lax.fori_loop(0, OUTER, step, 0)

    plsc.subcore_barrier()

    for k in range(ROWS_PER_TILE // CHUNK):
        s = pl.ds(r0 + k * CHUNK, CHUNK)
        pltpu.sync_copy(acc.at[s], out_hbm.at[cid, s])


@functools.cache
def _get_sc_scatter():
    return pl.kernel(
        _sc_scatter_body,
        out_type=jax.ShapeDtypeStruct((NC, N_PAD, D), jnp.float32),
        mesh=plsc.VectorSubcoreMesh(
            core_axis_name="c", subcore_axis_name="s",
            num_cores=NC, num_subcores=NS),
        scratch_types=[
            pltpu.VMEM_SHARED((N_PAD, D), jnp.float32),
            pltpu.VMEM((KI, CHUNK), jnp.int32),
            pltpu.VMEM((KI, CHUNK), jnp.int32),
            pltpu.VMEM((CHUNK, D), jnp.float32),
            pltpu.VMEM((CHUNK, D), jnp.float32),
            pltpu.SemaphoreType.DMA,
            pltpu.SemaphoreType.DMA,
        ],
        name="gcn_sc_scatter",
    )


def _sc_deg_body(dst_hbm, deg_hbm, dacc, dst_st, ones, sem):
    cid = lax.axis_index("c")
    sid = lax.axis_index("s")
    wid = cid * NS + sid
    r0 = sid * ROWS_PER_TILE
    zero16 = jnp.zeros((16,), jnp.float32)
    ones16 = jnp.ones((16,), jnp.float32)

    def zfill(r, _):
        for c in range(D // 16):
            ones[r, pl.ds(c * 16, 16)] = zero16
        return 0
    lax.fori_loop(0, CHUNK, zfill, 0)
    for k in range(ROWS_PER_TILE // CHUNK):
        pltpu.sync_copy(ones, dacc.at[pl.ds(r0 + k * CHUNK, CHUNK)])

    def ofill(r, _):
        for c in range(D // 16):
            ones[r, pl.ds(c * 16, 16)] = ones16
        return 0
    lax.fori_loop(0, CHUNK, ofill, 0)

    plsc.subcore_barrier()

    def step(t, _):
        pltpu.sync_copy(dst_hbm.at[wid, pl.ds(t * KI, KI)], dst_st)
        for j in range(KI):
            pltpu.async_copy(ones, dacc.at[dst_st.at[j]], sem, add=True)
        for j in range(KI):
            pltpu.make_async_copy(ones, dacc.at[dst_st.at[j]], sem).wait()
        return 0
    lax.fori_loop(0, OUTER, step, 0)

    plsc.subcore_barrier()

    for k in range(ROWS_PER_TILE // CHUNK):
        s = pl.ds(r0 + k * CHUNK, CHUNK)
        pltpu.sync_copy(dacc.at[s], deg_hbm.at[cid, s])


@functools.cache
def _get_sc_deg():
    return pl.kernel(
        _sc_deg_body,
        out_type=jax.ShapeDtypeStruct((NC, N_PAD, D), jnp.float32),
        mesh=plsc.VectorSubcoreMesh(
            core_axis_name="c", subcore_axis_name="s",
            num_cores=NC, num_subcores=NS),
        scratch_types=[
            pltpu.VMEM_SHARED((N_PAD, D), jnp.float32),
            pltpu.VMEM((KI, CHUNK), jnp.int32),
            pltpu.VMEM((CHUNK, D), jnp.float32),
            pltpu.SemaphoreType.DMA,
        ],
        name="gcn_sc_deg",
    )


RB = 1000


def _tc_body(p_ref, deg_ref, h_ref, w_ref, wihT_ref, whhT_ref, bih_ref,
             bhh_ref, o_ref):
    hp = jax.lax.Precision.HIGHEST
    g = p_ref[0] + p_ref[1]
    deg = deg_ref[0, :, 0:1] + deg_ref[1, :, 0:1]
    deg = jnp.maximum(deg, 1.0)
    agg = jax.lax.dot(g / deg, w_ref[...], precision=hp)
    gi = jax.lax.dot(agg, wihT_ref[...], precision=hp) + bih_ref[...]
    h = h_ref[...]
    gh = jax.lax.dot(h, whhT_ref[...], precision=hp) + bhh_ref[...]
    r = jax.nn.sigmoid(gi[:, :D] + gh[:, :D])
    z = jax.nn.sigmoid(gi[:, D:2 * D] + gh[:, D:2 * D])
    n = jnp.tanh(gi[:, 2 * D:] + r * gh[:, 2 * D:])
    o_ref[...] = (1.0 - z) * n + z * h


def _tc_layer(p, degp, h, w, wihT, whhT, bih, bhh):
    return pl.pallas_call(
        _tc_body,
        grid=(N // RB,),
        in_specs=[
            pl.BlockSpec((NC, RB, D), lambda i: (0, i, 0)),
            pl.BlockSpec((NC, RB, D), lambda i: (0, i, 0)),
            pl.BlockSpec((RB, D), lambda i: (i, 0)),
            pl.BlockSpec((D, D), lambda i: (0, 0)),
            pl.BlockSpec((D, 3 * D), lambda i: (0, 0)),
            pl.BlockSpec((D, 3 * D), lambda i: (0, 0)),
            pl.BlockSpec((1, 3 * D), lambda i: (0, 0)),
            pl.BlockSpec((1, 3 * D), lambda i: (0, 0)),
        ],
        out_specs=pl.BlockSpec((RB, D), lambda i: (i, 0)),
        out_shape=jax.ShapeDtypeStruct((N, D), jnp.float32),
    )(p, degp, h, w, wihT, whhT, bih, bhh)


def kernel(x, adj, edge, weight, W_ih, W_hh, b_ih, b_hh):
    src = edge[0].astype(jnp.int32)
    dst = edge[1].astype(jnp.int32)
    e = src.shape[0]
    pad = E_PAD - e
    src_p = jnp.concatenate([src, jnp.zeros((pad,), jnp.int32)])
    dst_p = jnp.concatenate([dst, jnp.full((pad,), DUMMY, jnp.int32)])
    src3 = src_p.reshape(NW, OUTER * KI, CHUNK)
    dst3 = dst_p.reshape(NW, OUTER * KI, CHUNK)
    wihT = W_ih.T
    whhT = W_hh.T
    bih2 = b_ih.reshape(1, 3 * D)
    bhh2 = b_hh.reshape(1, 3 * D)

    degp = _get_sc_deg()(dst3)
    h = x
    for i in range(3):
        p = _get_sc_scatter()(h, src3, dst3)
        h = _tc_layer(p, degp, h, weight[i], wihT, whhT, bih2, bhh2)
    return h


# speedup vs baseline: 3.2838x; 1.0687x over previous
import functools

import jax
import jax.numpy as jnp
from jax import lax
from jax.experimental import pallas as pl
from jax.experimental.pallas import tpu as pltpu
from jax.experimental.pallas import tpu_sc as plsc

N = 10000
D = 128
NC = 2
NS = 16
NW = NC * NS
CHUNK = 128
KI = 8
G0 = 16
G1 = 4
GMAX = max(G0, G1)
GROUP = KI * CHUNK
E0 = NS * G0 * GROUP
E1 = NS * G1 * GROUP
E_PAD = E0 + E1
N_PAD = 10240
ROWS_PER_TILE = N_PAD // NS
DUMMY = N


def _sc_scatter_body(h_hbm, src_hbm, dst_hbm, out_hbm,
                     acc, src_st, dst_st, rows_a, rows_b, sem_a, sem_b):
    cid = lax.axis_index("c")
    sid = lax.axis_index("s")
    wid = cid * NS + sid
    r0 = sid * ROWS_PER_TILE
    ng = jnp.where(cid == 0, G0, G1)
    zero16 = jnp.zeros((16,), jnp.float32)

    def zfill(r, _):
        for c in range(D // 16):
            rows_a[r, pl.ds(c * 16, 16)] = zero16
        return 0
    lax.fori_loop(0, CHUNK, zfill, 0)
    for k in range(ROWS_PER_TILE // CHUNK):
        pltpu.sync_copy(rows_a, acc.at[pl.ds(r0 + k * CHUNK, CHUNK)])

    plsc.subcore_barrier()

    bufs = (rows_a, rows_b)
    sems = (sem_a, sem_b)

    def step(t, _):
        pltpu.sync_copy(src_hbm.at[wid, pl.ds(t * KI, KI)], src_st)
        pltpu.sync_copy(dst_hbm.at[wid, pl.ds(t * KI, KI)], dst_st)
        pltpu.async_copy(h_hbm.at[src_st.at[0]], bufs[0], sems[0])
        for j in range(KI):
            cur = j % 2
            nxt = 1 - cur
            pltpu.make_async_copy(h_hbm.at[src_st.at[j]], bufs[cur],
                                  sems[cur]).wait()
            if j + 1 < KI:
                pltpu.async_copy(h_hbm.at[src_st.at[j + 1]], bufs[nxt],
                                 sems[nxt])
            pltpu.sync_copy(bufs[cur], acc.at[dst_st.at[j]], add=True)
        return 0
    lax.fori_loop(0, ng, step, 0)

    plsc.subcore_barrier()

    for k in range(ROWS_PER_TILE // CHUNK):
        s = pl.ds(r0 + k * CHUNK, CHUNK)
        pltpu.sync_copy(acc.at[s], out_hbm.at[cid, s])


@functools.cache
def _get_sc_scatter():
    return pl.kernel(
        _sc_scatter_body,
        out_type=jax.ShapeDtypeStruct((NC, N_PAD, D), jnp.float32),
        mesh=plsc.VectorSubcoreMesh(
            core_axis_name="c", subcore_axis_name="s",
            num_cores=NC, num_subcores=NS),
        scratch_types=[
            pltpu.VMEM_SHARED((N_PAD, D), jnp.float32),
            pltpu.VMEM((KI, CHUNK), jnp.int32),
            pltpu.VMEM((KI, CHUNK), jnp.int32),
            pltpu.VMEM((CHUNK, D), jnp.float32),
            pltpu.VMEM((CHUNK, D), jnp.float32),
            pltpu.SemaphoreType.DMA,
            pltpu.SemaphoreType.DMA,
        ],
        name="gcn_sc_scatter",
    )


def _sc_deg_body(dst_hbm, deg_hbm, dacc, dst_st, ones, sem):
    cid = lax.axis_index("c")
    sid = lax.axis_index("s")
    wid = cid * NS + sid
    r0 = sid * ROWS_PER_TILE
    ng = jnp.where(cid == 0, G0, G1)
    zero16 = jnp.zeros((16,), jnp.float32)
    ones16 = jnp.ones((16,), jnp.float32)

    def zfill(r, _):
        for c in range(D // 16):
            ones[r, pl.ds(c * 16, 16)] = zero16
        return 0
    lax.fori_loop(0, CHUNK, zfill, 0)
    for k in range(ROWS_PER_TILE // CHUNK):
        pltpu.sync_copy(ones, dacc.at[pl.ds(r0 + k * CHUNK, CHUNK)])

    def ofill(r, _):
        for c in range(D // 16):
            ones[r, pl.ds(c * 16, 16)] = ones16
        return 0
    lax.fori_loop(0, CHUNK, ofill, 0)

    plsc.subcore_barrier()

    def step(t, _):
        pltpu.sync_copy(dst_hbm.at[wid, pl.ds(t * KI, KI)], dst_st)
        for j in range(KI):
            pltpu.async_copy(ones, dacc.at[dst_st.at[j]], sem, add=True)
        for j in range(KI):
            pltpu.make_async_copy(ones, dacc.at[dst_st.at[j]], sem).wait()
        return 0
    lax.fori_loop(0, ng, step, 0)

    plsc.subcore_barrier()

    for k in range(ROWS_PER_TILE // CHUNK):
        s = pl.ds(r0 + k * CHUNK, CHUNK)
        pltpu.sync_copy(dacc.at[s], deg_hbm.at[cid, s])


@functools.cache
def _get_sc_deg():
    return pl.kernel(
        _sc_deg_body,
        out_type=jax.ShapeDtypeStruct((NC, N_PAD, D), jnp.float32),
        mesh=plsc.VectorSubcoreMesh(
            core_axis_name="c", subcore_axis_name="s",
            num_cores=NC, num_subcores=NS),
        scratch_types=[
            pltpu.VMEM_SHARED((N_PAD, D), jnp.float32),
            pltpu.VMEM((KI, CHUNK), jnp.int32),
            pltpu.VMEM((CHUNK, D), jnp.float32),
            pltpu.SemaphoreType.DMA,
        ],
        name="gcn_sc_deg",
    )


RB = 1000


def _tc_body(p_ref, deg_ref, h_ref, w_ref, wihT_ref, whhT_ref, bih_ref,
             bhh_ref, o_ref):
    hp = jax.lax.Precision.HIGHEST
    g = p_ref[0] + p_ref[1]
    deg = jnp.maximum(deg_ref[...], 1.0)
    agg = jax.lax.dot(g / deg, w_ref[...], precision=hp)
    gi = jax.lax.dot(agg, wihT_ref[...], precision=hp) + bih_ref[...]
    h = h_ref[...]
    gh = jax.lax.dot(h, whhT_ref[...], precision=hp) + bhh_ref[...]
    r = jax.nn.sigmoid(gi[:, :D] + gh[:, :D])
    z = jax.nn.sigmoid(gi[:, D:2 * D] + gh[:, D:2 * D])
    n = jnp.tanh(gi[:, 2 * D:] + r * gh[:, 2 * D:])
    o_ref[...] = (1.0 - z) * n + z * h


def _tc_layer(p, deg, h, w, wihT, whhT, bih, bhh):
    return pl.pallas_call(
        _tc_body,
        grid=(N // RB,),
        in_specs=[
            pl.BlockSpec((NC, RB, D), lambda i: (0, i, 0)),
            pl.BlockSpec((RB, 1), lambda i: (i, 0)),
            pl.BlockSpec((RB, D), lambda i: (i, 0)),
            pl.BlockSpec((D, D), lambda i: (0, 0)),
            pl.BlockSpec((D, 3 * D), lambda i: (0, 0)),
            pl.BlockSpec((D, 3 * D), lambda i: (0, 0)),
            pl.BlockSpec((1, 3 * D), lambda i: (0, 0)),
            pl.BlockSpec((1, 3 * D), lambda i: (0, 0)),
        ],
        out_specs=pl.BlockSpec((RB, D), lambda i: (i, 0)),
        out_shape=jax.ShapeDtypeStruct((N, D), jnp.float32),
    )(p, deg, h, w, wihT, whhT, bih, bhh)


def _split_edges(v, fill):
    pad = E_PAD - v.shape[0]
    vp = jnp.concatenate([v, jnp.full((pad,), fill, jnp.int32)])
    a0 = vp[:E0].reshape(NS, G0 * KI, CHUNK)
    a1 = vp[E0:].reshape(NS, G1 * KI, CHUNK)
    a1 = jnp.pad(a1, ((0, 0), (0, (GMAX - G1) * KI), (0, 0)))
    return jnp.concatenate([a0, a1], axis=0)


def kernel(x, adj, edge, weight, W_ih, W_hh, b_ih, b_hh):
    src3 = _split_edges(edge[0].astype(jnp.int32), 0)
    dst3 = _split_edges(edge[1].astype(jnp.int32), DUMMY)
    wihT = W_ih.T
    whhT = W_hh.T
    bih2 = b_ih.reshape(1, 3 * D)
    bhh2 = b_hh.reshape(1, 3 * D)

    degp = _get_sc_deg()(dst3)
    deg = degp[0, :, 0:1] + degp[1, :, 0:1]
    h = x
    for i in range(3):
        p = _get_sc_scatter()(h, src3, dst3)
        h = _tc_layer(p, deg, h, weight[i], wihT, whhT, bih2, bhh2)
    return h
